# baseline (speedup 1.0000x reference)
import jax
import jax.numpy as jnp
from jax import lax
from jax.experimental import pallas as pl
from jax.experimental.pallas import tpu as pltpu
from jax.experimental.pallas import tpu_sc as plsc

_VOCAB = 100000
_D = 1024
_B = 4
_L = 2048
_N = _B * _L

_NC = 2
_NS = 16
_NW = _NC * _NS
_RPW = _N // _NW
_CH = 32
_NCH = _RPW // _CH

_ROWS = 1024
_NB = 3


def _gather_body(ids_hbm, table_hbm, out_hbm, idx_v, buf0, buf1, buf2,
                 gsem0, gsem1, gsem2, osem0, osem1, osem2):
    wid = lax.axis_index("s") * _NC + lax.axis_index("c")
    base = wid * _RPW
    pltpu.sync_copy(ids_hbm.at[wid], idx_v)
    bufs = (buf0, buf1, buf2)
    gsems = (gsem0, gsem1, gsem2)
    osems = (osem0, osem1, osem2)
    ghandles = [None] * _NB
    ohandles = [None] * _NB
    for p in range(min(2, _NCH)):
        ghandles[p % _NB] = pltpu.async_copy(
            table_hbm.at[idx_v.at[p]], bufs[p % _NB], gsems[p % _NB])
    for c in range(_NCH):
        s = c % _NB
        n = c + 2
        if n < _NCH:
            sn = n % _NB
            if ohandles[sn] is not None:
                ohandles[sn].wait()
                ohandles[sn] = None
            ghandles[sn] = pltpu.async_copy(
                table_hbm.at[idx_v.at[n]], bufs[sn], gsems[sn])
        ghandles[s].wait()
        ohandles[s] = pltpu.async_copy(
            bufs[s], out_hbm.at[pl.ds(base + c * _CH, _CH)], osems[s])
    for h in ohandles:
        if h is not None:
            h.wait()


@jax.jit
def _gather(ids3, table):
    mesh = plsc.VectorSubcoreMesh(core_axis_name="c", subcore_axis_name="s")
    return pl.kernel(
        _gather_body,
        mesh=mesh,
        compiler_params=pltpu.CompilerParams(use_tc_tiling_on_sc=True),
        out_type=jax.ShapeDtypeStruct((_N, _D), jnp.float32),
        scratch_types=[
            pltpu.VMEM((_NCH, _CH), jnp.int32),
            pltpu.VMEM((_CH, _D), jnp.float32),
            pltpu.VMEM((_CH, _D), jnp.float32),
            pltpu.VMEM((_CH, _D), jnp.float32),
            pltpu.SemaphoreType.DMA,
            pltpu.SemaphoreType.DMA,
            pltpu.SemaphoreType.DMA,
            pltpu.SemaphoreType.DMA,
            pltpu.SemaphoreType.DMA,
            pltpu.SemaphoreType.DMA,
        ],
    )(ids3, table)


def _dense_body(e_ref, pos_ref, gam_ref, beta_ref, out_ref):
    e = e_ref[...] + pos_ref[...]
    ones = jnp.ones((_D, 1), jnp.float32)
    s1 = jax.lax.dot(e, ones, precision=jax.lax.Precision.HIGHEST)
    sq = jax.lax.dot(e * e, ones, precision=jax.lax.Precision.HIGHEST)
    mu = s1 * (1.0 / _D)
    var = jnp.maximum(sq * (1.0 / _D) - mu * mu, 0.0)
    y = (e - mu) * lax.rsqrt(var + 1e-5) * gam_ref[...] + beta_ref[...]
    n2 = jax.lax.dot(y * y, ones, precision=jax.lax.Precision.HIGHEST)
    nrm = jnp.sqrt(n2)
    nrmc = jnp.maximum(nrm, 1e-8)
    vn = jnp.minimum(nrmc, 2.0)
    ex = jnp.exp(vn)
    iex = 1.0 / ex
    xs = y * ((0.5 * (ex - iex)) / nrmc)
    t = 0.5 * (ex + iex)
    out_ref[...] = jnp.concatenate([t, xs], axis=1)


_PB = _L // _ROWS
_dense_call = pl.pallas_call(
    _dense_body,
    grid=(_PB, _B),
    in_specs=[
        pl.BlockSpec((_ROWS, _D), lambda i, j: (j * _PB + i, 0)),
        pl.BlockSpec((_ROWS, _D), lambda i, j: (i, 0)),
        pl.BlockSpec((1, _D), lambda i, j: (0, 0)),
        pl.BlockSpec((1, _D), lambda i, j: (0, 0)),
    ],
    out_specs=pl.BlockSpec((_ROWS, _D + 1), lambda i, j: (j * _PB + i, 0)),
    out_shape=jax.ShapeDtypeStruct((_N, _D + 1), jnp.float32),
)


def kernel(input_ids, token_table, pos_table, ln_gamma, ln_beta):
    Bp, Lp = input_ids.shape
    ids3 = input_ids.astype(jnp.int32).reshape(_NW, _NCH, _CH)
    gathered = _gather(ids3, token_table)
    x = _dense_call(gathered, pos_table[:Lp],
                    ln_gamma.reshape(1, _D), ln_beta.reshape(1, _D))
    return x.reshape(Bp, Lp, _D + 1)


# speedup vs baseline: 2.1006x; 2.1006x over previous
import jax
import jax.numpy as jnp
from jax import lax
from jax.experimental import pallas as pl
from jax.experimental.pallas import tpu as pltpu
from jax.experimental.pallas import tpu_sc as plsc

_VOCAB = 100000
_D = 1024
_B = 4
_L = 2048
_N = _B * _L

_NC = 2
_NS = 16
_NW = _NC * _NS
_RPW = _N // _NW
_CH = 32
_NCH = _RPW // _CH

_ROWS = 2048
_NB = 3


def _gather_body(ids_hbm, table_hbm, out_hbm, idx_v, buf0, buf1, buf2,
                 gsem0, gsem1, gsem2, osem0, osem1, osem2):
    wid = lax.axis_index("s") * _NC + lax.axis_index("c")
    base = wid * _RPW
    pltpu.sync_copy(ids_hbm.at[wid], idx_v)
    bufs = (buf0, buf1, buf2)
    gsems = (gsem0, gsem1, gsem2)
    osems = (osem0, osem1, osem2)
    ghandles = [None] * _NB
    ohandles = [None] * _NB
    for p in range(min(2, _NCH)):
        ghandles[p % _NB] = pltpu.async_copy(
            table_hbm.at[idx_v.at[p]], bufs[p % _NB], gsems[p % _NB])
    for c in range(_NCH):
        s = c % _NB
        n = c + 2
        if n < _NCH:
            sn = n % _NB
            if ohandles[sn] is not None:
                ohandles[sn].wait()
                ohandles[sn] = None
            ghandles[sn] = pltpu.async_copy(
                table_hbm.at[idx_v.at[n]], bufs[sn], gsems[sn])
        ghandles[s].wait()
        ohandles[s] = pltpu.async_copy(
            bufs[s], out_hbm.at[pl.ds(base + c * _CH, _CH)], osems[s])
    for h in ohandles:
        if h is not None:
            h.wait()


@jax.jit
def _gather(ids3, table):
    mesh = plsc.VectorSubcoreMesh(core_axis_name="c", subcore_axis_name="s")
    return pl.kernel(
        _gather_body,
        mesh=mesh,
        compiler_params=pltpu.CompilerParams(use_tc_tiling_on_sc=True),
        out_type=jax.ShapeDtypeStruct((_N, _D), jnp.float32),
        scratch_types=[
            pltpu.VMEM((_NCH, _CH), jnp.int32),
            pltpu.VMEM((_CH, _D), jnp.float32),
            pltpu.VMEM((_CH, _D), jnp.float32),
            pltpu.VMEM((_CH, _D), jnp.float32),
            pltpu.SemaphoreType.DMA,
            pltpu.SemaphoreType.DMA,
            pltpu.SemaphoreType.DMA,
            pltpu.SemaphoreType.DMA,
            pltpu.SemaphoreType.DMA,
            pltpu.SemaphoreType.DMA,
        ],
    )(ids3, table)


def _dense_body(e_ref, pos_ref, gam_ref, beta_ref, out_ref):
    e = e_ref[...] + pos_ref[...]
    s1 = jnp.sum(e, axis=1, keepdims=True)
    sq = jnp.sum(e * e, axis=1, keepdims=True)
    mu = s1 * (1.0 / _D)
    var = jnp.maximum(sq * (1.0 / _D) - mu * mu, 0.0)
    y = (e - mu) * lax.rsqrt(var + 1e-5) * gam_ref[...] + beta_ref[...]
    n2 = jnp.sum(y * y, axis=1, keepdims=True)
    nrm = jnp.sqrt(n2)
    nrmc = jnp.maximum(nrm, 1e-8)
    vn = jnp.minimum(nrmc, 2.0)
    ex = jnp.exp(vn)
    iex = 1.0 / ex
    xs = y * ((0.5 * (ex - iex)) / nrmc)
    t = 0.5 * (ex + iex)
    out_ref[...] = jnp.concatenate([t, xs], axis=1)


_PB = _L // _ROWS
_dense_call = pl.pallas_call(
    _dense_body,
    grid=(_PB, _B),
    in_specs=[
        pl.BlockSpec((_ROWS, _D), lambda i, j: (j * _PB + i, 0)),
        pl.BlockSpec((_ROWS, _D), lambda i, j: (i, 0)),
        pl.BlockSpec((1, _D), lambda i, j: (0, 0)),
        pl.BlockSpec((1, _D), lambda i, j: (0, 0)),
    ],
    out_specs=pl.BlockSpec((_ROWS, _D + 1), lambda i, j: (j * _PB + i, 0)),
    out_shape=jax.ShapeDtypeStruct((_N, _D + 1), jnp.float32),
)


def kernel(input_ids, token_table, pos_table, ln_gamma, ln_beta):
    Bp, Lp = input_ids.shape
    ids3 = input_ids.astype(jnp.int32).reshape(_NW, _NCH, _CH)
    gathered = _gather(ids3, token_table)
    x = _dense_call(gathered, pos_table[:Lp],
                    ln_gamma.reshape(1, _D), ln_beta.reshape(1, _D))
    return x.reshape(Bp, Lp, _D + 1)
